# baseline (speedup 1.0000x reference)
import jax
import jax.numpy as jnp
from jax.experimental import pallas as pl
from jax.experimental.pallas import tpu as pltpu


def _masked_copy_kernel(x_ref, o_ref, zeros_vmem, sem_zero, sem_copy):
    nb = x_ref.shape[0]
    for i in range(1, nb):
        pltpu.make_async_copy(x_ref.at[i], o_ref.at[i], sem_copy.at[i - 1]).start()
    zeros_vmem[...] = jnp.zeros_like(zeros_vmem)
    copy_zero = pltpu.make_async_copy(zeros_vmem, o_ref.at[0], sem_zero)
    copy_zero.start()
    for i in range(1, nb):
        pltpu.make_async_copy(x_ref.at[i], o_ref.at[i], sem_copy.at[i - 1]).wait()
    copy_zero.wait()


def kernel(time_images_season_list):
    x = time_images_season_list
    _, b, t, c, n = x.shape
    x2 = x.reshape(b, t, c * n)
    out = pl.pallas_call(
        _masked_copy_kernel,
        in_specs=[pl.BlockSpec(memory_space=pl.ANY)],
        out_specs=pl.BlockSpec(memory_space=pl.ANY),
        out_shape=jax.ShapeDtypeStruct((b, t, c * n), x.dtype),
        scratch_shapes=[
            pltpu.VMEM((t, c * n), x.dtype),
            pltpu.SemaphoreType.DMA,
            pltpu.SemaphoreType.DMA((b - 1,)),
        ],
    )(x2)
    return out.reshape(b, t, c, n)


# speedup vs baseline: 12.9321x; 12.9321x over previous
import jax
import jax.numpy as jnp
from jax.experimental import pallas as pl


_TB = 2048


def _masked_copy_kernel(x_ref, o_ref):
    b = pl.program_id(0)

    @pl.when(b == 0)
    def _zero():
        o_ref[...] = jnp.zeros_like(o_ref)

    @pl.when(b != 0)
    def _copy():
        o_ref[...] = x_ref[...]


def kernel(time_images_season_list):
    x = time_images_season_list
    _, b, t, c, n = x.shape
    x2 = x.reshape(b, t, c * n)
    out = pl.pallas_call(
        _masked_copy_kernel,
        grid=(b, t // _TB),
        in_specs=[pl.BlockSpec((1, _TB, c * n), lambda i, j: (i, j, 0))],
        out_specs=pl.BlockSpec((1, _TB, c * n), lambda i, j: (i, j, 0)),
        out_shape=jax.ShapeDtypeStruct((b, t, c * n), x.dtype),
    )(x2)
    return out.reshape(b, t, c, n)


# speedup vs baseline: 12.9890x; 1.0044x over previous
import jax
import jax.numpy as jnp
from jax.experimental import pallas as pl
from jax.experimental.pallas import tpu as pltpu


_TB = 512
_SLOTS = 8
_LOOK = 4


def _masked_copy_kernel(x_ref, o_ref, slots, zeros_vmem, in_sems, out_sems, zero_sems):
    nb, t, w = x_ref.shape
    nchunk = t // _TB
    nblk = (nb - 1) * nchunk

    def src(j):
        bb = 1 + j // nchunk
        cc = j % nchunk
        return x_ref.at[bb, pl.ds(cc * _TB, _TB), :]

    def dst(j):
        bb = 1 + j // nchunk
        cc = j % nchunk
        return o_ref.at[bb, pl.ds(cc * _TB, _TB), :]

    zeros_vmem[...] = jnp.zeros_like(zeros_vmem)
    for cc in range(nchunk):
        pltpu.make_async_copy(
            zeros_vmem, o_ref.at[0, pl.ds(cc * _TB, _TB), :], zero_sems.at[cc]
        ).start()

    for j in range(min(_LOOK, nblk)):
        pltpu.make_async_copy(src(j), slots.at[j % _SLOTS], in_sems.at[j % _SLOTS]).start()
    for j in range(nblk):
        s = j % _SLOTS
        pltpu.make_async_copy(src(j), slots.at[s], in_sems.at[s]).wait()
        pltpu.make_async_copy(slots.at[s], dst(j), out_sems.at[s]).start()
        jn = j + _LOOK
        if jn < nblk:
            sn = jn % _SLOTS
            if jn >= _SLOTS:
                pltpu.make_async_copy(
                    slots.at[sn], dst(jn - _SLOTS), out_sems.at[sn]
                ).wait()
            pltpu.make_async_copy(src(jn), slots.at[sn], in_sems.at[sn]).start()
    for j in range(max(0, nblk - _SLOTS), nblk):
        s = j % _SLOTS
        pltpu.make_async_copy(slots.at[s], dst(j), out_sems.at[s]).wait()
    for cc in range(nchunk):
        pltpu.make_async_copy(
            zeros_vmem, o_ref.at[0, pl.ds(cc * _TB, _TB), :], zero_sems.at[cc]
        ).wait()


def kernel(time_images_season_list):
    x = time_images_season_list
    _, b, t, c, n = x.shape
    x2 = x.reshape(b, t, c * n)
    out = pl.pallas_call(
        _masked_copy_kernel,
        in_specs=[pl.BlockSpec(memory_space=pl.ANY)],
        out_specs=pl.BlockSpec(memory_space=pl.ANY),
        out_shape=jax.ShapeDtypeStruct((b, t, c * n), x.dtype),
        scratch_shapes=[
            pltpu.VMEM((_SLOTS, _TB, c * n), x.dtype),
            pltpu.VMEM((_TB, c * n), x.dtype),
            pltpu.SemaphoreType.DMA((_SLOTS,)),
            pltpu.SemaphoreType.DMA((_SLOTS,)),
            pltpu.SemaphoreType.DMA((t // _TB,)),
        ],
    )(x2)
    return out.reshape(b, t, c, n)
